# Initial kernel scaffold; baseline (speedup 1.0000x reference)
import jax
import jax.numpy as jnp
from jax.experimental import pallas as pl


def kernel(node_attr, batch_idx, edge_index, strats_spd, atom_emb, summary_emb, W_spd_enc, Wq, bq, Wk, bk, Wv, bv, Wa, ba, Wspd, Wlin, blin, gn, bn, go, bo, Wfin, bfin):
    raise NotImplementedError("write your pallas kernel here")


# speedup vs baseline: 1.2198x; 1.2198x over previous
import functools

import jax
import jax.numpy as jnp
import numpy as np
from jax.experimental import pallas as pl
from jax.experimental.pallas import tpu as pltpu

N = 10000
E = 160000
H = 8
D = 128
DK = D // H
L = 2
NOUT = 128
BN = 1000
AW = 176


def _gelu(x):
    return x * 0.5 * (1.0 + jax.lax.erf(x * np.float32(1.0 / np.sqrt(2.0))))


def _ln(x, g, b):
    m = jnp.mean(x, axis=-1, keepdims=True)
    v = jnp.mean((x - m) ** 2, axis=-1, keepdims=True)
    return (x - m) * jax.lax.rsqrt(v + 1e-5) * g + b


def _enc_body(attr_ref, dmat_ref, c_ref, h_ref):
    h_ref[...] = (
        jnp.dot(attr_ref[...], dmat_ref[...], preferred_element_type=jnp.float32)
        + c_ref[...]
    )


def _enc(attr_f, dmat, c):
    return pl.pallas_call(
        _enc_body,
        grid=(N // BN,),
        in_specs=[
            pl.BlockSpec((BN, 9), lambda i: (i, 0)),
            pl.BlockSpec((9, D), lambda i: (0, 0)),
            pl.BlockSpec((1, D), lambda i: (0, 0)),
        ],
        out_specs=pl.BlockSpec((BN, D), lambda i: (i, 0)),
        out_shape=jax.ShapeDtypeStruct((N, D), jnp.float32),
    )(attr_f, dmat, c)


def _pre_body(h_ref, wq_ref, bq_ref, wk_ref, bk_ref, wv_ref, bv_ref, mk_ref,
              qcat_ref, k_ref, v_ref):
    h = h_ref[...]
    qn = jnp.dot(h, wq_ref[...], preferred_element_type=jnp.float32) + bq_ref[...]
    kn = jnp.dot(h, wk_ref[...], preferred_element_type=jnp.float32) + bk_ref[...]
    vn = jnp.dot(h, wv_ref[...], preferred_element_type=jnp.float32) + bv_ref[...]
    gq = jnp.dot(qn, mk_ref[...], preferred_element_type=jnp.float32)
    qcat_ref[:, :D] = qn
    qcat_ref[:, D:] = gq
    k_ref[...] = kn
    v_ref[...] = vn


def _pre(h, wq, bq, wk, bk, wv, bv, mk):
    full = lambda r, c: pl.BlockSpec((r, c), lambda i: (0, 0))
    return pl.pallas_call(
        _pre_body,
        grid=(N // BN,),
        in_specs=[
            pl.BlockSpec((BN, D), lambda i: (i, 0)),
            full(D, D), full(1, D), full(D, D), full(1, D), full(D, D),
            full(1, D), full(D, 32),
        ],
        out_specs=[
            pl.BlockSpec((BN, D + 32), lambda i: (i, 0)),
            pl.BlockSpec((BN, D), lambda i: (i, 0)),
            pl.BlockSpec((BN, D), lambda i: (i, 0)),
        ],
        out_shape=[
            jax.ShapeDtypeStruct((N, D + 32), jnp.float32),
            jax.ShapeDtypeStruct((N, D), jnp.float32),
            jax.ShapeDtypeStruct((N, D), jnp.float32),
        ],
    )(h, wq, bq, wk, bk, wv, bv, mk)


def _post_body(a0_ref, a1_ref, h_ref, mv_ref, r16_ref, wa_ref, ba_ref, gn_ref,
               bn_ref, wlin_ref, blin_ref, go_ref, bo_ref, hout_ref):
    a = a0_ref[...] + a1_ref[...]
    num = a[:, :D] + jnp.dot(a[:, D:D + 32], mv_ref[...],
                             preferred_element_type=jnp.float32)
    srep = jnp.dot(a[:, D + 32:], r16_ref[...],
                   preferred_element_type=jnp.float32)
    aggr = num / (srep + 1e-16)
    h = h_ref[...]
    t = (jnp.dot(_gelu(aggr), wa_ref[...], preferred_element_type=jnp.float32)
         + ba_ref[...] + h)
    t = _ln(t, gn_ref[...], bn_ref[...])
    t2 = (jnp.dot(_gelu(t), wlin_ref[...], preferred_element_type=jnp.float32)
          + blin_ref[...] + t)
    hout_ref[...] = _ln(t2, go_ref[...], bo_ref[...])


def _post(a0, a1, h, mv, r16, wa, ba, gn, bn, wlin, blin, go, bo):
    full = lambda r, c: pl.BlockSpec((r, c), lambda i: (0, 0))
    return pl.pallas_call(
        _post_body,
        grid=(N // BN,),
        in_specs=[
            pl.BlockSpec((BN, AW), lambda i: (i, 0)),
            pl.BlockSpec((BN, AW), lambda i: (i, 0)),
            pl.BlockSpec((BN, D), lambda i: (i, 0)),
            full(32, D), full(16, D), full(D, D), full(1, D), full(1, D),
            full(1, D), full(D, D), full(1, D), full(1, D), full(1, D),
        ],
        out_specs=pl.BlockSpec((BN, D), lambda i: (i, 0)),
        out_shape=jax.ShapeDtypeStruct((N, D), jnp.float32),
    )(a0, a1, h, mv, r16, wa, ba, gn, bn, wlin, blin, go, bo)


def _fin_body(h_ref, w_ref, b_ref, o_ref):
    o_ref[...] = (jnp.dot(h_ref[...], w_ref[...],
                          preferred_element_type=jnp.float32) + b_ref[...])


def _fin(h, w, b):
    return pl.pallas_call(
        _fin_body,
        grid=(N // BN,),
        in_specs=[
            pl.BlockSpec((BN, D), lambda i: (i, 0)),
            pl.BlockSpec((D, NOUT), lambda i: (0, 0)),
            pl.BlockSpec((1, NOUT), lambda i: (0, 0)),
        ],
        out_specs=pl.BlockSpec((BN, NOUT), lambda i: (i, 0)),
        out_shape=jax.ShapeDtypeStruct((N, NOUT), jnp.float32),
    )(h, w, b)


def _edge_phase(qcat, kn, vn, src, dst, strats):
    q_e = jnp.take(qcat, dst, axis=0)
    k_e = jnp.take(kn, src, axis=0)
    logit = jnp.sum((q_e[:, :D] * k_e).reshape(E, H, DK), axis=-1)
    logit = logit + jnp.sum(
        q_e[:, D:].reshape(E, H, 4) * strats[:, None, :], axis=-1)
    p = jnp.exp(logit)
    v_e = jnp.take(vn, src, axis=0).reshape(E, H, DK)
    num = jax.ops.segment_sum((p[:, :, None] * v_e).reshape(E, D), dst,
                              num_segments=N)
    tmat = jax.ops.segment_sum((p[:, :, None] * strats[:, None, :]).reshape(E, 32),
                               dst, num_segments=N)
    s = jax.ops.segment_sum(p, dst, num_segments=N)
    return jnp.concatenate([num, tmat, s, jnp.zeros((N, 8), jnp.float32)], axis=1)


def kernel(node_attr, batch_idx, edge_index, strats_spd, atom_emb, summary_emb,
           W_spd_enc, Wq, bq, Wk, bk, Wv, bv, Wa, ba, Wspd, Wlin, blin, gn, bn,
           go, bo, Wfin, bfin):
    del batch_idx, summary_emb
    dmat = (atom_emb[:, 1, :] - atom_emb[:, 0, :])
    cvec = jnp.sum(atom_emb[:, 0, :], axis=0)[None, :]
    attr_f = node_attr.astype(jnp.float32)

    src = edge_index[0]
    dst = edge_index[1]

    d_ids = jnp.arange(D)
    c32 = jnp.arange(32)
    r16 = jnp.where((d_ids[None, :] // DK) == jnp.arange(16)[:, None],
                    1.0, 0.0).astype(jnp.float32)

    h = _enc(attr_f, dmat, cvec)
    for l in range(L):
        ck = W_spd_enc @ Wspd[l] @ Wk[l]
        cv = W_spd_enc @ Wspd[l] @ Wv[l]
        mk = jnp.where((c32[None, :] // 4) == (d_ids[:, None] // DK),
                       ck.T[:, c32 % 4], 0.0)
        mv = jnp.where((d_ids[None, :] // DK) == (c32[:, None] // 4),
                       cv[c32 % 4, :], 0.0)
        qcat, kn, vn = _pre(h, Wq[l] * np.float32(0.25),
                            bq[l][None] * np.float32(0.25), Wk[l],
                            bk[l][None], Wv[l], bv[l][None], mk)
        acc = _edge_phase(qcat, kn, vn, src, dst, strats_spd)
        zero = jnp.zeros_like(acc)
        h = _post(acc, zero, h, mv, r16, Wa[l], ba[l][None], gn[l][None],
                  bn[l][None], Wlin[l], blin[l][None], go[l][None], bo[l][None])
    return _fin(h, Wfin, bfin[None])


# speedup vs baseline: 1.7566x; 1.4401x over previous
import functools

import jax
import jax.numpy as jnp
import numpy as np
from jax import lax
from jax.experimental import pallas as pl
from jax.experimental.pallas import tpu as pltpu
from jax.experimental.pallas import tpu_sc as plsc

N = 10000
E = 160000
H = 8
D = 128
DK = D // H
L = 2
NOUT = 128
BN = 1000
AW = 176
QW = D + 32

NP = 10240
CB = 128
NCH = 40
NW = 32
EP = NW * NCH * CB
RPW = NP // 16


def _gelu(x):
    return x * 0.5 * (1.0 + jax.lax.erf(x * np.float32(1.0 / np.sqrt(2.0))))


def _ln(x, g, b):
    m = jnp.mean(x, axis=-1, keepdims=True)
    v = jnp.mean((x - m) ** 2, axis=-1, keepdims=True)
    return (x - m) * jax.lax.rsqrt(v + 1e-5) * g + b


def _enc_body(attr_ref, dmat_ref, c_ref, h_ref):
    h_ref[...] = (
        jnp.dot(attr_ref[...], dmat_ref[...], preferred_element_type=jnp.float32)
        + c_ref[...]
    )


def _enc(attr_f, dmat, c):
    return pl.pallas_call(
        _enc_body,
        grid=(N // BN,),
        in_specs=[
            pl.BlockSpec((BN, 9), lambda i: (i, 0)),
            pl.BlockSpec((9, D), lambda i: (0, 0)),
            pl.BlockSpec((1, D), lambda i: (0, 0)),
        ],
        out_specs=pl.BlockSpec((BN, D), lambda i: (i, 0)),
        out_shape=jax.ShapeDtypeStruct((N, D), jnp.float32),
    )(attr_f, dmat, c)


def _pre_body(h_ref, wq_ref, bq_ref, wk_ref, bk_ref, wv_ref, bv_ref, mk_ref,
              qcat_ref, k_ref, v_ref):
    h = h_ref[...]
    qn = jnp.dot(h, wq_ref[...], preferred_element_type=jnp.float32) + bq_ref[...]
    kn = jnp.dot(h, wk_ref[...], preferred_element_type=jnp.float32) + bk_ref[...]
    vn = jnp.dot(h, wv_ref[...], preferred_element_type=jnp.float32) + bv_ref[...]
    gq = jnp.dot(qn, mk_ref[...], preferred_element_type=jnp.float32)
    qcat_ref[:, :D] = qn
    qcat_ref[:, D:] = gq
    k_ref[...] = kn
    v_ref[...] = vn


def _pre(h, wq, bq, wk, bk, wv, bv, mk):
    full = lambda r, c: pl.BlockSpec((r, c), lambda i: (0, 0))
    return pl.pallas_call(
        _pre_body,
        grid=(N // BN,),
        in_specs=[
            pl.BlockSpec((BN, D), lambda i: (i, 0)),
            full(D, D), full(1, D), full(D, D), full(1, D), full(D, D),
            full(1, D), full(D, 32),
        ],
        out_specs=[
            pl.BlockSpec((BN, QW), lambda i: (i, 0)),
            pl.BlockSpec((BN, D), lambda i: (i, 0)),
            pl.BlockSpec((BN, D), lambda i: (i, 0)),
        ],
        out_shape=[
            jax.ShapeDtypeStruct((NP, QW), jnp.float32),
            jax.ShapeDtypeStruct((NP, D), jnp.float32),
            jax.ShapeDtypeStruct((NP, D), jnp.float32),
        ],
    )(h, wq, bq, wk, bk, wv, bv, mk)


def _post_body(v0_ref, v1_ref, t0_ref, t1_ref, h_ref, mv_ref, r16_ref, wa_ref,
               ba_ref, gn_ref, bn_ref, wlin_ref, blin_ref, go_ref, bo_ref,
               hout_ref):
    ts = t0_ref[...] + t1_ref[...]
    num = (v0_ref[...] + v1_ref[...]
           + jnp.dot(ts[:, :32], mv_ref[...],
                     preferred_element_type=jnp.float32))
    srep = jnp.dot(ts[:, 32:], r16_ref[...],
                   preferred_element_type=jnp.float32)
    aggr = num / (srep + 1e-16)
    h = h_ref[...]
    t = (jnp.dot(_gelu(aggr), wa_ref[...], preferred_element_type=jnp.float32)
         + ba_ref[...] + h)
    t = _ln(t, gn_ref[...], bn_ref[...])
    t2 = (jnp.dot(_gelu(t), wlin_ref[...], preferred_element_type=jnp.float32)
          + blin_ref[...] + t)
    hout_ref[...] = _ln(t2, go_ref[...], bo_ref[...])


def _post(v0, v1, t0, t1, h, mv, r16, wa, ba, gn, bn, wlin, blin, go, bo):
    full = lambda r, c: pl.BlockSpec((r, c), lambda i: (0, 0))
    return pl.pallas_call(
        _post_body,
        grid=(N // BN,),
        in_specs=[
            pl.BlockSpec((BN, D), lambda i: (i, 0)),
            pl.BlockSpec((BN, D), lambda i: (i, 0)),
            pl.BlockSpec((BN, SW), lambda i: (i, 0)),
            pl.BlockSpec((BN, SW), lambda i: (i, 0)),
            pl.BlockSpec((BN, D), lambda i: (i, 0)),
            full(32, D), full(16, D), full(D, D), full(1, D), full(1, D),
            full(1, D), full(D, D), full(1, D), full(1, D), full(1, D),
        ],
        out_specs=pl.BlockSpec((BN, D), lambda i: (i, 0)),
        out_shape=jax.ShapeDtypeStruct((N, D), jnp.float32),
    )(v0, v1, t0, t1, h, mv, r16, wa, ba, gn, bn, wlin, blin, go, bo)


def _fin_body(h_ref, w_ref, b_ref, o_ref):
    o_ref[...] = (jnp.dot(h_ref[...], w_ref[...],
                          preferred_element_type=jnp.float32) + b_ref[...])


def _fin(h, w, b):
    return pl.pallas_call(
        _fin_body,
        grid=(N // BN,),
        in_specs=[
            pl.BlockSpec((BN, D), lambda i: (i, 0)),
            pl.BlockSpec((D, NOUT), lambda i: (0, 0)),
            pl.BlockSpec((1, NOUT), lambda i: (0, 0)),
        ],
        out_specs=pl.BlockSpec((BN, NOUT), lambda i: (i, 0)),
        out_shape=jax.ShapeDtypeStruct((N, NOUT), jnp.float32),
    )(h, w, b)


_sc_mesh = plsc.VectorSubcoreMesh(core_axis_name="c", subcore_axis_name="s")
_sc_params = pltpu.CompilerParams(use_tc_tiling_on_sc=False,
                                  needs_layout_passes=False)
SW = 48


@functools.partial(
    pl.kernel,
    out_type=[jax.ShapeDtypeStruct((EP, H), jnp.float32),
              jax.ShapeDtypeStruct((NP, SW), jnp.float32),
              jax.ShapeDtypeStruct((NP, SW), jnp.float32)],
    mesh=_sc_mesh,
    compiler_params=_sc_params,
    scratch_types=[
        pltpu.VMEM_SHARED((NP, SW), jnp.float32),
        pltpu.VMEM((NCH, CB), jnp.int32),
        pltpu.VMEM((NCH, CB), jnp.int32),
        pltpu.VMEM((NCH, 4 * CB), jnp.float32),
        pltpu.VMEM((CB, QW), jnp.float32),
        pltpu.VMEM((CB, D), jnp.float32),
        pltpu.VMEM((CB, H), jnp.float32),
        pltpu.VMEM((CB, SW), jnp.float32),
        pltpu.SemaphoreType.DMA,
        pltpu.SemaphoreType.DMA,
    ],
)
def _edge_sc1(qcat_hbm, kn_hbm, dst_hbm, src_hbm, strat_hbm, zero_hbm,
              p_out, t_out0, t_out1, acc, dstv, srcv, stratv, qv, kv, pv, mv,
              sem_q, sem_k):
    cid = lax.axis_index("c")
    sid = lax.axis_index("s")
    w = sid * 2 + cid
    r0 = sid * RPW
    pltpu.sync_copy(zero_hbm.at[pl.ds(r0, RPW)], acc.at[pl.ds(r0, RPW)])
    pltpu.sync_copy(dst_hbm.at[pl.ds(w * NCH, NCH)], dstv)
    pltpu.sync_copy(src_hbm.at[pl.ds(w * NCH, NCH)], srcv)
    pltpu.sync_copy(strat_hbm.at[pl.ds(w * NCH, NCH)], stratv)
    plsc.subcore_barrier()

    lane = lax.iota(jnp.int32, 16)
    z16 = jnp.zeros((16,), jnp.int32)

    def chunk(ci, carry):
        idx_d = dstv.at[ci]
        idx_s = srcv.at[ci]
        cp_q = pltpu.async_copy(qcat_hbm.at[idx_d], qv, sem_q)
        cp_k = pltpu.async_copy(kn_hbm.at[idx_s], kv, sem_k)
        cp_q.wait()
        cp_k.wait()

        def stage_a(g, carry_a):
            erow = g * 16 + lane
            svecs = [plsc.load_gather(stratv, [z16 + ci, erow * 4 + j])
                     for j in range(4)]
            for h in range(H):
                a = jnp.zeros((16,), jnp.float32)
                for dk in range(DK):
                    col = z16 + (h * DK + dk)
                    a = a + (plsc.load_gather(qv, [erow, col])
                             * plsc.load_gather(kv, [erow, col]))
                for j in range(4):
                    gq = plsc.load_gather(qv, [erow, z16 + (D + h * 4 + j)])
                    a = a + gq * svecs[j]
                plsc.store_scatter(pv, [erow, z16 + h], jnp.exp(a))
            return carry_a
        lax.fori_loop(0, CB // 16, stage_a, 0)

        def stage_b(e, carry_b):
            erow = z16 + e
            jj = lane % 4
            hh = lane // 4
            sb = plsc.load_gather(stratv, [z16 + ci, e * 4 + jj])
            pb0 = plsc.load_gather(pv, [erow, hh])
            plsc.store_scatter(mv, [erow, lane], pb0 * sb)
            pb1 = plsc.load_gather(pv, [erow, 4 + hh])
            plsc.store_scatter(mv, [erow, 16 + lane], pb1 * sb)
            ps = plsc.load_gather(pv, [erow, jnp.minimum(lane, 7)])
            ps = jnp.where(lane < 8, ps, jnp.float32(0.0))
            plsc.store_scatter(mv, [erow, 32 + lane], ps)
            return carry_b
        lax.fori_loop(0, CB, stage_b, 0)

        pltpu.sync_copy(pv, p_out.at[pl.ds((w * NCH + ci) * CB, CB)])
        pltpu.sync_copy(mv, acc.at[idx_d], add=True)
        return carry
    lax.fori_loop(0, NCH, chunk, 0)

    plsc.subcore_barrier()

    @pl.when(cid == 0)
    def _():
        pltpu.sync_copy(acc.at[pl.ds(r0, RPW)], t_out0.at[pl.ds(r0, RPW)])

    @pl.when(cid == 1)
    def _():
        pltpu.sync_copy(acc.at[pl.ds(r0, RPW)], t_out1.at[pl.ds(r0, RPW)])


@functools.partial(
    pl.kernel,
    out_type=[jax.ShapeDtypeStruct((NP, D), jnp.float32),
              jax.ShapeDtypeStruct((NP, D), jnp.float32)],
    mesh=_sc_mesh,
    compiler_params=_sc_params,
    scratch_types=[
        pltpu.VMEM_SHARED((NP, D), jnp.float32),
        pltpu.VMEM((NCH, CB), jnp.int32),
        pltpu.VMEM((NCH, CB), jnp.int32),
        pltpu.VMEM((CB, D), jnp.float32),
        pltpu.VMEM((CB, H), jnp.float32),
        pltpu.VMEM((CB, D), jnp.float32),
        pltpu.SemaphoreType.DMA,
    ],
)
def _edge_sc2(vn_hbm, dst_hbm, src_hbm, p_hbm, zero_hbm,
              v_out0, v_out1, acc, dstv, srcv, vv, pv, mv, sem_v):
    cid = lax.axis_index("c")
    sid = lax.axis_index("s")
    w = sid * 2 + cid
    r0 = sid * RPW
    pltpu.sync_copy(zero_hbm.at[pl.ds(r0, RPW)], acc.at[pl.ds(r0, RPW)])
    pltpu.sync_copy(dst_hbm.at[pl.ds(w * NCH, NCH)], dstv)
    pltpu.sync_copy(src_hbm.at[pl.ds(w * NCH, NCH)], srcv)
    plsc.subcore_barrier()

    lane = lax.iota(jnp.int32, 16)
    z16 = jnp.zeros((16,), jnp.int32)

    def chunk(ci, carry):
        idx_d = dstv.at[ci]
        idx_s = srcv.at[ci]
        cp_v = pltpu.async_copy(vn_hbm.at[idx_s], vv, sem_v)
        pltpu.sync_copy(p_hbm.at[pl.ds((w * NCH + ci) * CB, CB)], pv)
        cp_v.wait()

        def stage_b(e, carry_b):
            erow = z16 + e
            for h in range(H):
                vvec = plsc.load_gather(vv, [erow, h * DK + lane])
                pb = plsc.load_gather(pv, [erow, z16 + h])
                plsc.store_scatter(mv, [erow, h * DK + lane], vvec * pb)
            return carry_b
        lax.fori_loop(0, CB, stage_b, 0)

        pltpu.sync_copy(mv, acc.at[idx_d], add=True)
        return carry
    lax.fori_loop(0, NCH, chunk, 0)

    plsc.subcore_barrier()

    @pl.when(cid == 0)
    def _():
        pltpu.sync_copy(acc.at[pl.ds(r0, RPW)], v_out0.at[pl.ds(r0, RPW)])

    @pl.when(cid == 1)
    def _():
        pltpu.sync_copy(acc.at[pl.ds(r0, RPW)], v_out1.at[pl.ds(r0, RPW)])


def kernel(node_attr, batch_idx, edge_index, strats_spd, atom_emb, summary_emb,
           W_spd_enc, Wq, bq, Wk, bk, Wv, bv, Wa, ba, Wspd, Wlin, blin, gn, bn,
           go, bo, Wfin, bfin):
    del batch_idx, summary_emb
    dmat = (atom_emb[:, 1, :] - atom_emb[:, 0, :])
    cvec = jnp.sum(atom_emb[:, 0, :], axis=0)[None, :]
    attr_f = node_attr.astype(jnp.float32)

    src = edge_index[0]
    dst = edge_index[1]

    d_ids = jnp.arange(D)
    c32 = jnp.arange(32)
    r16 = jnp.where((d_ids[None, :] // DK) == jnp.arange(16)[:, None],
                    1.0, 0.0).astype(jnp.float32)

    pad_e = EP - E
    i32 = jnp.int32
    dst_p = jnp.concatenate(
        [dst.astype(i32), jnp.full((pad_e,), N, i32)]).reshape(EP // CB, CB)
    src_p = jnp.concatenate(
        [src.astype(i32), jnp.zeros((pad_e,), i32)]).reshape(EP // CB, CB)
    strat_p = jnp.concatenate(
        [strats_spd, jnp.zeros((pad_e, 4), jnp.float32)]).reshape(EP // CB, 4 * CB)
    zero_s = jnp.zeros((NP, SW), jnp.float32)
    zero_v = jnp.zeros((NP, D), jnp.float32)

    h = _enc(attr_f, dmat, cvec)
    for l in range(L):
        ck = W_spd_enc @ Wspd[l] @ Wk[l]
        cv = W_spd_enc @ Wspd[l] @ Wv[l]
        mk = jnp.where((c32[None, :] // 4) == (d_ids[:, None] // DK),
                       ck.T[:, c32 % 4], 0.0)
        mv = jnp.where((d_ids[None, :] // DK) == (c32[:, None] // 4),
                       cv[c32 % 4, :], 0.0)
        qcat, kn, vn = _pre(h, Wq[l] * np.float32(0.25),
                            bq[l][None] * np.float32(0.25), Wk[l],
                            bk[l][None], Wv[l], bv[l][None], mk)
        p_e, t0, t1 = _edge_sc1(qcat, kn, dst_p, src_p, strat_p, zero_s)
        v0, v1 = _edge_sc2(vn, dst_p, src_p, p_e, zero_v)
        h = _post(v0, v1, t0, t1, h, mv, r16, Wa[l], ba[l][None], gn[l][None],
                  bn[l][None], Wlin[l], blin[l][None], go[l][None], bo[l][None])
    return _fin(h, Wfin, bfin[None])


# speedup vs baseline: 2.4215x; 1.3785x over previous
import functools

import jax
import jax.numpy as jnp
import numpy as np
from jax import lax
from jax.experimental import pallas as pl
from jax.experimental.pallas import tpu as pltpu
from jax.experimental.pallas import tpu_sc as plsc

N = 10000
E = 160000
H = 8
D = 128
DK = D // H
L = 2
NOUT = 128
BN = 1000
AW = 176
QW = D + 32

NP = 10240
CB = 128
NCH = 40
NW = 32
EP = NW * NCH * CB
RPW = NP // 16


def _gelu(x):
    return x * 0.5 * (1.0 + jax.lax.erf(x * np.float32(1.0 / np.sqrt(2.0))))


def _ln(x, g, b):
    m = jnp.mean(x, axis=-1, keepdims=True)
    v = jnp.mean((x - m) ** 2, axis=-1, keepdims=True)
    return (x - m) * jax.lax.rsqrt(v + 1e-5) * g + b


def _enc_body(attr_ref, dmat_ref, c_ref, h_ref):
    h_ref[...] = (
        jnp.dot(attr_ref[...], dmat_ref[...], preferred_element_type=jnp.float32)
        + c_ref[...]
    )


def _enc(attr_f, dmat, c):
    return pl.pallas_call(
        _enc_body,
        grid=(N // BN,),
        in_specs=[
            pl.BlockSpec((BN, 9), lambda i: (i, 0)),
            pl.BlockSpec((9, D), lambda i: (0, 0)),
            pl.BlockSpec((1, D), lambda i: (0, 0)),
        ],
        out_specs=pl.BlockSpec((BN, D), lambda i: (i, 0)),
        out_shape=jax.ShapeDtypeStruct((N, D), jnp.float32),
    )(attr_f, dmat, c)


def _pre_body(h_ref, wq_ref, bq_ref, wk_ref, bk_ref, wv_ref, bv_ref, mk_ref,
              qcat_ref, k_ref, v_ref):
    h = h_ref[...]
    qn = jnp.dot(h, wq_ref[...], preferred_element_type=jnp.float32) + bq_ref[...]
    kn = jnp.dot(h, wk_ref[...], preferred_element_type=jnp.float32) + bk_ref[...]
    vn = jnp.dot(h, wv_ref[...], preferred_element_type=jnp.float32) + bv_ref[...]
    gq = jnp.dot(qn, mk_ref[...], preferred_element_type=jnp.float32)
    qcat_ref[:, :D] = qn
    qcat_ref[:, D:] = gq
    k_ref[...] = kn
    v_ref[...] = vn


def _pre(h, wq, bq, wk, bk, wv, bv, mk):
    full = lambda r, c: pl.BlockSpec((r, c), lambda i: (0, 0))
    return pl.pallas_call(
        _pre_body,
        grid=(N // BN,),
        in_specs=[
            pl.BlockSpec((BN, D), lambda i: (i, 0)),
            full(D, D), full(1, D), full(D, D), full(1, D), full(D, D),
            full(1, D), full(D, 32),
        ],
        out_specs=[
            pl.BlockSpec((BN, QW), lambda i: (i, 0)),
            pl.BlockSpec((BN, D), lambda i: (i, 0)),
            pl.BlockSpec((BN, D), lambda i: (i, 0)),
        ],
        out_shape=[
            jax.ShapeDtypeStruct((NP, QW), jnp.float32),
            jax.ShapeDtypeStruct((NP, D), jnp.float32),
            jax.ShapeDtypeStruct((NP, D), jnp.float32),
        ],
    )(h, wq, bq, wk, bk, wv, bv, mk)


def _post_body(v0_ref, v1_ref, t0_ref, t1_ref, h_ref, mv_ref, r16_ref, wa_ref,
               ba_ref, gn_ref, bn_ref, wlin_ref, blin_ref, go_ref, bo_ref,
               hout_ref):
    ts = t0_ref[...] + t1_ref[...]
    num = (v0_ref[...] + v1_ref[...]
           + jnp.dot(ts[:, :32], mv_ref[...],
                     preferred_element_type=jnp.float32))
    srep = jnp.dot(ts[:, 32:], r16_ref[...],
                   preferred_element_type=jnp.float32)
    aggr = num / (srep + 1e-16)
    h = h_ref[...]
    t = (jnp.dot(_gelu(aggr), wa_ref[...], preferred_element_type=jnp.float32)
         + ba_ref[...] + h)
    t = _ln(t, gn_ref[...], bn_ref[...])
    t2 = (jnp.dot(_gelu(t), wlin_ref[...], preferred_element_type=jnp.float32)
          + blin_ref[...] + t)
    hout_ref[...] = _ln(t2, go_ref[...], bo_ref[...])


def _post(v0, v1, t0, t1, h, mv, r16, wa, ba, gn, bn, wlin, blin, go, bo):
    full = lambda r, c: pl.BlockSpec((r, c), lambda i: (0, 0))
    return pl.pallas_call(
        _post_body,
        grid=(N // BN,),
        in_specs=[
            pl.BlockSpec((BN, D), lambda i: (i, 0)),
            pl.BlockSpec((BN, D), lambda i: (i, 0)),
            pl.BlockSpec((BN, SW), lambda i: (i, 0)),
            pl.BlockSpec((BN, SW), lambda i: (i, 0)),
            pl.BlockSpec((BN, D), lambda i: (i, 0)),
            full(32, D), full(16, D), full(D, D), full(1, D), full(1, D),
            full(1, D), full(D, D), full(1, D), full(1, D), full(1, D),
        ],
        out_specs=pl.BlockSpec((BN, D), lambda i: (i, 0)),
        out_shape=jax.ShapeDtypeStruct((N, D), jnp.float32),
    )(v0, v1, t0, t1, h, mv, r16, wa, ba, gn, bn, wlin, blin, go, bo)


def _fin_body(h_ref, w_ref, b_ref, o_ref):
    o_ref[...] = (jnp.dot(h_ref[...], w_ref[...],
                          preferred_element_type=jnp.float32) + b_ref[...])


def _fin(h, w, b):
    return pl.pallas_call(
        _fin_body,
        grid=(N // BN,),
        in_specs=[
            pl.BlockSpec((BN, D), lambda i: (i, 0)),
            pl.BlockSpec((D, NOUT), lambda i: (0, 0)),
            pl.BlockSpec((1, NOUT), lambda i: (0, 0)),
        ],
        out_specs=pl.BlockSpec((BN, NOUT), lambda i: (i, 0)),
        out_shape=jax.ShapeDtypeStruct((N, NOUT), jnp.float32),
    )(h, w, b)


_sc_mesh = plsc.VectorSubcoreMesh(core_axis_name="c", subcore_axis_name="s")
_sc_params = pltpu.CompilerParams(use_tc_tiling_on_sc=False,
                                  needs_layout_passes=False)
SW = 48


@functools.partial(
    pl.kernel,
    out_type=[jax.ShapeDtypeStruct((EP, H), jnp.float32),
              jax.ShapeDtypeStruct((NP, SW), jnp.float32),
              jax.ShapeDtypeStruct((NP, SW), jnp.float32)],
    mesh=_sc_mesh,
    compiler_params=_sc_params,
    scratch_types=[
        pltpu.VMEM_SHARED((NP, SW), jnp.float32),
        pltpu.VMEM((4, CB), jnp.int32),
        pltpu.VMEM((4, CB), jnp.int32),
        pltpu.VMEM((4, CB, 4), jnp.float32),
        pltpu.VMEM((2, CB, QW), jnp.float32),
        pltpu.VMEM((2, CB, D), jnp.float32),
        pltpu.VMEM((2, CB, H), jnp.float32),
        pltpu.VMEM((2, CB, SW), jnp.float32),
    ] + [pltpu.SemaphoreType.DMA] * 20,
)
def _edge_sc1(qcat_hbm, kn_hbm, dst_hbm, src_hbm, strat_hbm, zero_hbm,
              p_out, t_out0, t_out1, acc, dstv, srcv, stratv, qv, kv, pv, mv,
              *sems):
    semd = sems[0:4]
    semsr = sems[4:8]
    semt = sems[8:12]
    semq = sems[12:14]
    semk = sems[14:16]
    semp = sems[16:18]
    semm = sems[18:20]
    cid = lax.axis_index("c")
    sid = lax.axis_index("s")
    w = sid * 2 + cid
    r0 = sid * RPW
    pltpu.sync_copy(zero_hbm.at[pl.ds(r0, RPW)], acc.at[pl.ds(r0, RPW)])
    plsc.subcore_barrier()

    lane = lax.iota(jnp.int32, 16)
    z16 = jnp.zeros((16,), jnp.int32)
    c0 = w * NCH

    def meta_copies(ci, slot):
        return (pltpu.make_async_copy(dst_hbm.at[c0 + ci], dstv.at[slot],
                                      semd[slot]),
                pltpu.make_async_copy(src_hbm.at[c0 + ci], srcv.at[slot],
                                      semsr[slot]),
                pltpu.make_async_copy(strat_hbm.at[c0 + ci], stratv.at[slot],
                                      semt[slot]))

    def gather_copies(b, slot):
        return (pltpu.make_async_copy(qcat_hbm.at[dstv.at[slot]], qv.at[b],
                                      semq[b]),
                pltpu.make_async_copy(kn_hbm.at[srcv.at[slot]], kv.at[b],
                                      semk[b]))

    def out_copies(ci, b, slot):
        return (pltpu.make_async_copy(
                    pv.at[b], p_out.at[pl.ds((c0 + ci) * CB, CB)], semp[b]),
                pltpu.make_async_copy(mv.at[b], acc.at[dstv.at[slot]],
                                      semm[b]))

    for cd in meta_copies(0, 0) + meta_copies(1, 1):
        cd.start()
    cd0, cs0, _ = meta_copies(0, 0)
    cd0.wait()
    cs0.wait()
    for cg in gather_copies(0, 0):
        cg.start()

    def quad(qi, carry):
        for u in range(4):
            ci = qi * 4 + u
            b = u % 2
            slot_n = (u + 1) % 4
            slot_p = (u + 2) % 4
            for cg in gather_copies(b, u):
                cg.wait()
            if u < 2:
                @pl.when(qi > 0)
                def _(b=b, u=u):
                    cp, cm = out_copies(0, b, u)
                    cp.wait()
                    cm.wait()
            else:
                cp, cm = out_copies(0, b, u)
                cp.wait()
                cm.wait()
            if u < 2:
                for cd in meta_copies(ci + 2, slot_p):
                    cd.start()
            else:
                @pl.when(qi < NCH // 4 - 1)
                def _(ci=ci, slot_p=slot_p):
                    for cd in meta_copies(ci + 2, slot_p):
                        cd.start()
            def start_next(slot_n=slot_n, b=b):
                cdn, csn, _ = meta_copies(0, slot_n)
                cdn.wait()
                csn.wait()
                for cg in gather_copies(1 - b, slot_n):
                    cg.start()
            if u < 3:
                start_next()
            else:
                pl.when(qi < NCH // 4 - 1)(start_next)
            _, _, ct = meta_copies(0, u)
            ct.wait()

            def stage_a(g, carry_a, b=b, u=u):
                erow = g * 16 + lane
                svecs = [plsc.load_gather(stratv, [z16 + u, erow, z16 + j])
                         for j in range(4)]
                for h in range(H):
                    a = jnp.zeros((16,), jnp.float32)
                    for dk in range(DK):
                        col = z16 + (h * DK + dk)
                        a = a + (plsc.load_gather(qv, [z16 + b, erow, col])
                                 * plsc.load_gather(kv, [z16 + b, erow, col]))
                    for j in range(4):
                        gq = plsc.load_gather(
                            qv, [z16 + b, erow, z16 + (D + h * 4 + j)])
                        a = a + gq * svecs[j]
                    plsc.store_scatter(pv, [z16 + b, erow, z16 + h],
                                       jnp.exp(a))
                return carry_a
            lax.fori_loop(0, CB // 16, stage_a, 0)

            def stage_b(e, carry_b, b=b, u=u):
                erow = z16 + e
                jj = lane % 4
                hh = lane // 4
                sb = plsc.load_gather(stratv, [z16 + u, erow, jj])
                pb0 = plsc.load_gather(pv, [z16 + b, erow, hh])
                plsc.store_scatter(mv, [z16 + b, erow, lane], pb0 * sb)
                pb1 = plsc.load_gather(pv, [z16 + b, erow, 4 + hh])
                plsc.store_scatter(mv, [z16 + b, erow, 16 + lane], pb1 * sb)
                ps = plsc.load_gather(pv, [z16 + b, erow, jnp.minimum(lane, 7)])
                ps = jnp.where(lane < 8, ps, jnp.float32(0.0))
                plsc.store_scatter(mv, [z16 + b, erow, 32 + lane], ps)
                return carry_b
            lax.fori_loop(0, CB, stage_b, 0)

            cp, cm = out_copies(ci, b, u)
            cp.start()
            cm.start(add=True)
        return carry
    lax.fori_loop(0, NCH // 4, quad, 0)

    for b in range(2):
        cp, cm = out_copies(0, b, 2 + b)
        cp.wait()
        cm.wait()

    plsc.subcore_barrier()

    @pl.when(cid == 0)
    def _():
        pltpu.sync_copy(acc.at[pl.ds(r0, RPW)], t_out0.at[pl.ds(r0, RPW)])

    @pl.when(cid == 1)
    def _():
        pltpu.sync_copy(acc.at[pl.ds(r0, RPW)], t_out1.at[pl.ds(r0, RPW)])


CB2 = 64
NCH2 = EP // (NW * CB2)


@functools.partial(
    pl.kernel,
    out_type=[jax.ShapeDtypeStruct((NP, D), jnp.float32),
              jax.ShapeDtypeStruct((NP, D), jnp.float32)],
    mesh=_sc_mesh,
    compiler_params=_sc_params,
    scratch_types=[
        pltpu.VMEM_SHARED((NP, D), jnp.float32),
        pltpu.VMEM((4, CB2), jnp.int32),
        pltpu.VMEM((4, CB2), jnp.int32),
        pltpu.VMEM((4, CB2, H), jnp.float32),
        pltpu.VMEM((2, CB2, D), jnp.float32),
        pltpu.VMEM((2, CB2, D), jnp.float32),
    ] + [pltpu.SemaphoreType.DMA] * 16,
)
def _edge_sc2(vn_hbm, dst_hbm, src_hbm, p_hbm, zero_hbm,
              v_out0, v_out1, acc, dstv, srcv, pvr, vv, mv, *sems):
    semd = sems[0:4]
    semsr = sems[4:8]
    semt = sems[8:12]
    semv = sems[12:14]
    semm = sems[14:16]
    cid = lax.axis_index("c")
    sid = lax.axis_index("s")
    w = sid * 2 + cid
    r0 = sid * RPW
    pltpu.sync_copy(zero_hbm.at[pl.ds(r0, RPW)], acc.at[pl.ds(r0, RPW)])
    plsc.subcore_barrier()

    lane = lax.iota(jnp.int32, 16)
    z16 = jnp.zeros((16,), jnp.int32)
    c0 = w * NCH2

    def meta_copies(ci, slot):
        return (pltpu.make_async_copy(dst_hbm.at[c0 + ci], dstv.at[slot],
                                      semd[slot]),
                pltpu.make_async_copy(src_hbm.at[c0 + ci], srcv.at[slot],
                                      semsr[slot]),
                pltpu.make_async_copy(p_hbm.at[pl.ds((c0 + ci) * CB2, CB2)],
                                      pvr.at[slot], semt[slot]))

    def gather_copies(b, slot):
        return (pltpu.make_async_copy(vn_hbm.at[srcv.at[slot]], vv.at[b],
                                      semv[b]),)

    def out_copies(b, slot):
        return (pltpu.make_async_copy(mv.at[b], acc.at[dstv.at[slot]],
                                      semm[b]),)

    for cd in meta_copies(0, 0) + meta_copies(1, 1):
        cd.start()
    _, cs0, _ = meta_copies(0, 0)
    cs0.wait()
    for cg in gather_copies(0, 0):
        cg.start()

    def quad(qi, carry):
        for u in range(4):
            ci = qi * 4 + u
            b = u % 2
            slot_n = (u + 1) % 4
            slot_p = (u + 2) % 4
            for cg in gather_copies(b, u):
                cg.wait()
            if u < 2:
                @pl.when(qi > 0)
                def _(b=b, u=u):
                    for cm in out_copies(b, u):
                        cm.wait()
            else:
                for cm in out_copies(b, u):
                    cm.wait()
            if u < 2:
                for cd in meta_copies(ci + 2, slot_p):
                    cd.start()
            else:
                @pl.when(qi < NCH2 // 4 - 1)
                def _(ci=ci, slot_p=slot_p):
                    for cd in meta_copies(ci + 2, slot_p):
                        cd.start()

            def start_next(slot_n=slot_n, b=b):
                _, csn, _ = meta_copies(0, slot_n)
                csn.wait()
                for cg in gather_copies(1 - b, slot_n):
                    cg.start()
            if u < 3:
                start_next()
            else:
                pl.when(qi < NCH2 // 4 - 1)(start_next)

            _, _, ct = meta_copies(0, u)
            ct.wait()

            def stage_b(e, carry_b, b=b, u=u):
                erow = z16 + e
                for h in range(H):
                    vvec = plsc.load_gather(vv, [z16 + b, erow, h * DK + lane])
                    pb = plsc.load_gather(pvr, [z16 + u, erow, z16 + h])
                    plsc.store_scatter(mv, [z16 + b, erow, h * DK + lane],
                                       vvec * pb)
                return carry_b
            lax.fori_loop(0, CB2, stage_b, 0)

            cdw, _, _ = meta_copies(0, u)
            cdw.wait()
            for cm in out_copies(b, u):
                cm.start(add=True)
        return carry
    lax.fori_loop(0, NCH2 // 4, quad, 0)

    for b in range(2):
        for cm in out_copies(b, 2 + b):
            cm.wait()

    plsc.subcore_barrier()

    @pl.when(cid == 0)
    def _():
        pltpu.sync_copy(acc.at[pl.ds(r0, RPW)], v_out0.at[pl.ds(r0, RPW)])

    @pl.when(cid == 1)
    def _():
        pltpu.sync_copy(acc.at[pl.ds(r0, RPW)], v_out1.at[pl.ds(r0, RPW)])


def kernel(node_attr, batch_idx, edge_index, strats_spd, atom_emb, summary_emb,
           W_spd_enc, Wq, bq, Wk, bk, Wv, bv, Wa, ba, Wspd, Wlin, blin, gn, bn,
           go, bo, Wfin, bfin):
    del batch_idx, summary_emb
    dmat = (atom_emb[:, 1, :] - atom_emb[:, 0, :])
    cvec = jnp.sum(atom_emb[:, 0, :], axis=0)[None, :]
    attr_f = node_attr.astype(jnp.float32)

    src = edge_index[0]
    dst = edge_index[1]

    d_ids = jnp.arange(D)
    c32 = jnp.arange(32)
    r16 = jnp.where((d_ids[None, :] // DK) == jnp.arange(16)[:, None],
                    1.0, 0.0).astype(jnp.float32)

    pad_e = EP - E
    i32 = jnp.int32
    dst_p = jnp.concatenate(
        [dst.astype(i32), jnp.full((pad_e,), N, i32)]).reshape(EP // CB, CB)
    src_p = jnp.concatenate(
        [src.astype(i32), jnp.zeros((pad_e,), i32)]).reshape(EP // CB, CB)
    strat_p = jnp.concatenate(
        [strats_spd, jnp.zeros((pad_e, 4), jnp.float32)]).reshape(EP // CB, CB, 4)
    dst_p2 = dst_p.reshape(EP // CB2, CB2)
    src_p2 = src_p.reshape(EP // CB2, CB2)
    zero_s = jnp.zeros((NP, SW), jnp.float32)
    zero_v = jnp.zeros((NP, D), jnp.float32)

    h = _enc(attr_f, dmat, cvec)
    for l in range(L):
        ck = W_spd_enc @ Wspd[l] @ Wk[l]
        cv = W_spd_enc @ Wspd[l] @ Wv[l]
        mk = jnp.where((c32[None, :] // 4) == (d_ids[:, None] // DK),
                       ck.T[:, c32 % 4], 0.0)
        mv = jnp.where((d_ids[None, :] // DK) == (c32[:, None] // 4),
                       cv[c32 % 4, :], 0.0)
        qcat, kn, vn = _pre(h, Wq[l] * np.float32(0.25),
                            bq[l][None] * np.float32(0.25), Wk[l],
                            bk[l][None], Wv[l], bv[l][None], mk)
        p_e, t0, t1 = _edge_sc1(qcat, kn, dst_p, src_p, strat_p, zero_s)
        v0, v1 = _edge_sc2(vn, dst_p2, src_p2, p_e, zero_v)
        h = _post(v0, v1, t0, t1, h, mv, r16, Wa[l], ba[l][None], gn[l][None],
                  bn[l][None], Wlin[l], blin[l][None], go[l][None], bo[l][None])
    return _fin(h, Wfin, bfin[None])


# speedup vs baseline: 3.1064x; 1.2829x over previous
import functools

import jax
import jax.numpy as jnp
import numpy as np
from jax import lax
from jax.experimental import pallas as pl
from jax.experimental.pallas import tpu as pltpu
from jax.experimental.pallas import tpu_sc as plsc

N = 10000
E = 160000
H = 8
D = 128
DK = D // H
L = 2
NOUT = 128
BN = 1000
AW = 176
QW = D + 32

NP = 10240
CB = 128
NCH = 40
NW = 32
EP = NW * NCH * CB
RPW = NP // 16


def _gelu(x):
    return x * 0.5 * (1.0 + jax.lax.erf(x * np.float32(1.0 / np.sqrt(2.0))))


def _ln(x, g, b):
    m = jnp.mean(x, axis=-1, keepdims=True)
    v = jnp.mean((x - m) ** 2, axis=-1, keepdims=True)
    return (x - m) * jax.lax.rsqrt(v + 1e-5) * g + b


def _enc_body(attr_ref, dmat_ref, c_ref, h_ref):
    h_ref[...] = (
        jnp.dot(attr_ref[...], dmat_ref[...], preferred_element_type=jnp.float32)
        + c_ref[...]
    )


def _enc(attr_f, dmat, c):
    return pl.pallas_call(
        _enc_body,
        grid=(N // BN,),
        in_specs=[
            pl.BlockSpec((BN, 9), lambda i: (i, 0)),
            pl.BlockSpec((9, D), lambda i: (0, 0)),
            pl.BlockSpec((1, D), lambda i: (0, 0)),
        ],
        out_specs=pl.BlockSpec((BN, D), lambda i: (i, 0)),
        out_shape=jax.ShapeDtypeStruct((N, D), jnp.float32),
    )(attr_f, dmat, c)


def _pre_body(h_ref, wq_ref, bq_ref, wk_ref, bk_ref, wv_ref, bv_ref, mk_ref,
              qcat_ref, k_ref, v_ref):
    h = h_ref[...]
    qn = jnp.dot(h, wq_ref[...], preferred_element_type=jnp.float32) + bq_ref[...]
    kn = jnp.dot(h, wk_ref[...], preferred_element_type=jnp.float32) + bk_ref[...]
    vn = jnp.dot(h, wv_ref[...], preferred_element_type=jnp.float32) + bv_ref[...]
    gq = jnp.dot(qn, mk_ref[...], preferred_element_type=jnp.float32)
    qcat_ref[:, :D] = qn
    qcat_ref[:, D:] = gq
    k_ref[...] = kn
    v_ref[...] = vn


def _pre(h, wq, bq, wk, bk, wv, bv, mk):
    full = lambda r, c: pl.BlockSpec((r, c), lambda i: (0, 0))
    return pl.pallas_call(
        _pre_body,
        grid=(N // BN,),
        in_specs=[
            pl.BlockSpec((BN, D), lambda i: (i, 0)),
            full(D, D), full(1, D), full(D, D), full(1, D), full(D, D),
            full(1, D), full(D, 32),
        ],
        out_specs=[
            pl.BlockSpec((BN, QW), lambda i: (i, 0)),
            pl.BlockSpec((BN, D), lambda i: (i, 0)),
            pl.BlockSpec((BN, D), lambda i: (i, 0)),
        ],
        out_shape=[
            jax.ShapeDtypeStruct((NP, QW), jnp.float32),
            jax.ShapeDtypeStruct((NP, D), jnp.float32),
            jax.ShapeDtypeStruct((NP, D), jnp.float32),
        ],
    )(h, wq, bq, wk, bk, wv, bv, mk)


def _post_body(v0_ref, v1_ref, t0_ref, t1_ref, h_ref, mv_ref, r16_ref, wa_ref,
               ba_ref, gn_ref, bn_ref, wlin_ref, blin_ref, go_ref, bo_ref,
               hout_ref):
    ts = t0_ref[...] + t1_ref[...]
    num = (v0_ref[...] + v1_ref[...]
           + jnp.dot(ts[:, :32], mv_ref[...],
                     preferred_element_type=jnp.float32))
    srep = jnp.dot(ts[:, 32:], r16_ref[...],
                   preferred_element_type=jnp.float32)
    aggr = num / (srep + 1e-16)
    h = h_ref[...]
    t = (jnp.dot(_gelu(aggr), wa_ref[...], preferred_element_type=jnp.float32)
         + ba_ref[...] + h)
    t = _ln(t, gn_ref[...], bn_ref[...])
    t2 = (jnp.dot(_gelu(t), wlin_ref[...], preferred_element_type=jnp.float32)
          + blin_ref[...] + t)
    hout_ref[...] = _ln(t2, go_ref[...], bo_ref[...])


def _post(v0, v1, t0, t1, h, mv, r16, wa, ba, gn, bn, wlin, blin, go, bo):
    full = lambda r, c: pl.BlockSpec((r, c), lambda i: (0, 0))
    return pl.pallas_call(
        _post_body,
        grid=(N // BN,),
        in_specs=[
            pl.BlockSpec((BN, D), lambda i: (i, 0)),
            pl.BlockSpec((BN, D), lambda i: (i, 0)),
            pl.BlockSpec((BN, SW), lambda i: (i, 0)),
            pl.BlockSpec((BN, SW), lambda i: (i, 0)),
            pl.BlockSpec((BN, D), lambda i: (i, 0)),
            full(32, D), full(16, D), full(D, D), full(1, D), full(1, D),
            full(1, D), full(D, D), full(1, D), full(1, D), full(1, D),
        ],
        out_specs=pl.BlockSpec((BN, D), lambda i: (i, 0)),
        out_shape=jax.ShapeDtypeStruct((N, D), jnp.float32),
    )(v0, v1, t0, t1, h, mv, r16, wa, ba, gn, bn, wlin, blin, go, bo)


def _fin_body(h_ref, w_ref, b_ref, o_ref):
    o_ref[...] = (jnp.dot(h_ref[...], w_ref[...],
                          preferred_element_type=jnp.float32) + b_ref[...])


def _fin(h, w, b):
    return pl.pallas_call(
        _fin_body,
        grid=(N // BN,),
        in_specs=[
            pl.BlockSpec((BN, D), lambda i: (i, 0)),
            pl.BlockSpec((D, NOUT), lambda i: (0, 0)),
            pl.BlockSpec((1, NOUT), lambda i: (0, 0)),
        ],
        out_specs=pl.BlockSpec((BN, NOUT), lambda i: (i, 0)),
        out_shape=jax.ShapeDtypeStruct((N, NOUT), jnp.float32),
    )(h, w, b)


_sc_mesh = plsc.VectorSubcoreMesh(core_axis_name="c", subcore_axis_name="s")
_sc_params = pltpu.CompilerParams(use_tc_tiling_on_sc=False,
                                  needs_layout_passes=False)
SW = 48


@functools.partial(
    pl.kernel,
    out_type=[jax.ShapeDtypeStruct((EP, H), jnp.float32),
              jax.ShapeDtypeStruct((NP, SW), jnp.float32),
              jax.ShapeDtypeStruct((NP, SW), jnp.float32)],
    mesh=_sc_mesh,
    compiler_params=_sc_params,
    scratch_types=[
        pltpu.VMEM_SHARED((NP, SW), jnp.float32),
        pltpu.VMEM((4, CB), jnp.int32),
        pltpu.VMEM((4, CB), jnp.int32),
        pltpu.VMEM((4, CB, 4), jnp.float32),
        pltpu.VMEM((2, CB, QW), jnp.float32),
        pltpu.VMEM((2, CB, D), jnp.float32),
        pltpu.VMEM((2, CB, H), jnp.float32),
        pltpu.VMEM((2, CB, SW), jnp.float32),
        pltpu.VMEM((16, QW + 1), jnp.float32),
        pltpu.VMEM((16, D + 1), jnp.float32),
    ] + [pltpu.SemaphoreType.DMA] * 20,
)
def _edge_sc1(qcat_hbm, kn_hbm, dst_hbm, src_hbm, strat_hbm, zero_hbm,
              p_out, t_out0, t_out1, acc, dstv, srcv, stratv, qv, kv, pv, mv,
              qp, kp, *sems):
    semd = sems[0:4]
    semsr = sems[4:8]
    semt = sems[8:12]
    semq = sems[12:14]
    semk = sems[14:16]
    semp = sems[16:18]
    semm = sems[18:20]
    cid = lax.axis_index("c")
    sid = lax.axis_index("s")
    w = sid * 2 + cid
    r0 = sid * RPW
    pltpu.sync_copy(zero_hbm.at[pl.ds(r0, RPW)], acc.at[pl.ds(r0, RPW)])
    plsc.subcore_barrier()

    lane = lax.iota(jnp.int32, 16)
    z16 = jnp.zeros((16,), jnp.int32)
    c0 = w * NCH

    def meta_copies(ci, slot):
        return (pltpu.make_async_copy(dst_hbm.at[c0 + ci], dstv.at[slot],
                                      semd[slot]),
                pltpu.make_async_copy(src_hbm.at[c0 + ci], srcv.at[slot],
                                      semsr[slot]),
                pltpu.make_async_copy(strat_hbm.at[c0 + ci], stratv.at[slot],
                                      semt[slot]))

    def gather_copies(b, slot):
        return (pltpu.make_async_copy(qcat_hbm.at[dstv.at[slot]], qv.at[b],
                                      semq[b]),
                pltpu.make_async_copy(kn_hbm.at[srcv.at[slot]], kv.at[b],
                                      semk[b]))

    def out_copies(ci, b, slot):
        return (pltpu.make_async_copy(
                    pv.at[b], p_out.at[pl.ds((c0 + ci) * CB, CB)], semp[b]),
                pltpu.make_async_copy(mv.at[b], acc.at[dstv.at[slot]],
                                      semm[b]))

    for cd in meta_copies(0, 0) + meta_copies(1, 1):
        cd.start()
    cd0, cs0, _ = meta_copies(0, 0)
    cd0.wait()
    cs0.wait()
    for cg in gather_copies(0, 0):
        cg.start()

    def quad(qi, carry):
        for u in range(4):
            ci = qi * 4 + u
            b = u % 2
            slot_n = (u + 1) % 4
            slot_p = (u + 2) % 4
            for cg in gather_copies(b, u):
                cg.wait()
            if u < 2:
                @pl.when(qi > 0)
                def _(b=b, u=u):
                    cp, cm = out_copies(0, b, u)
                    cp.wait()
                    cm.wait()
            else:
                cp, cm = out_copies(0, b, u)
                cp.wait()
                cm.wait()
            if u < 2:
                for cd in meta_copies(ci + 2, slot_p):
                    cd.start()
            else:
                @pl.when(qi < NCH // 4 - 1)
                def _(ci=ci, slot_p=slot_p):
                    for cd in meta_copies(ci + 2, slot_p):
                        cd.start()
            def start_next(slot_n=slot_n, b=b):
                cdn, csn, _ = meta_copies(0, slot_n)
                cdn.wait()
                csn.wait()
                for cg in gather_copies(1 - b, slot_n):
                    cg.start()
            if u < 3:
                start_next()
            else:
                pl.when(qi < NCH // 4 - 1)(start_next)
            _, _, ct = meta_copies(0, u)
            ct.wait()

            def stage_a(g, carry_a, b=b, u=u):
                erow = g * 16 + lane
                for r in range(16):
                    for c in range(QW // 16):
                        qp[r, pl.ds(c * 16, 16)] = (
                            qv[b, g * 16 + r, pl.ds(c * 16, 16)])
                    for c in range(D // 16):
                        kp[r, pl.ds(c * 16, 16)] = (
                            kv[b, g * 16 + r, pl.ds(c * 16, 16)])
                svecs = [plsc.load_gather(stratv, [z16 + u, erow, z16 + j])
                         for j in range(4)]

                def head(h, carry_h, b=b, erow=erow, svecs=svecs):
                    accs = [jnp.zeros((16,), jnp.float32) for _ in range(4)]
                    for dk in range(DK):
                        col = z16 + (h * DK + dk)
                        accs[dk % 4] = accs[dk % 4] + (
                            plsc.load_gather(qp, [lane, col])
                            * plsc.load_gather(kp, [lane, col]))
                    for j in range(4):
                        gq = plsc.load_gather(
                            qp, [lane, z16 + (D + h * 4 + j)])
                        accs[j] = accs[j] + gq * svecs[j]
                    a = (accs[0] + accs[1]) + (accs[2] + accs[3])
                    plsc.store_scatter(pv, [z16 + b, erow, z16 + h],
                                       jnp.exp(a))
                    return carry_h
                lax.fori_loop(0, H, head, 0)
                return carry_a
            lax.fori_loop(0, CB // 16, stage_a, 0)

            def stage_b(e, carry_b, b=b, u=u):
                erow = z16 + e
                jj = lane % 4
                hh = lane // 4
                sb = plsc.load_gather(stratv, [z16 + u, erow, jj])
                pb0 = plsc.load_gather(pv, [z16 + b, erow, hh])
                plsc.store_scatter(mv, [z16 + b, erow, lane], pb0 * sb)
                pb1 = plsc.load_gather(pv, [z16 + b, erow, 4 + hh])
                plsc.store_scatter(mv, [z16 + b, erow, 16 + lane], pb1 * sb)
                ps = plsc.load_gather(pv, [z16 + b, erow, jnp.minimum(lane, 7)])
                ps = jnp.where(lane < 8, ps, jnp.float32(0.0))
                plsc.store_scatter(mv, [z16 + b, erow, 32 + lane], ps)
                return carry_b
            lax.fori_loop(0, CB, stage_b, 0)

            cp, cm = out_copies(ci, b, u)
            cp.start()
            cm.start(add=True)
        return carry
    lax.fori_loop(0, NCH // 4, quad, 0)

    for b in range(2):
        cp, cm = out_copies(0, b, 2 + b)
        cp.wait()
        cm.wait()

    plsc.subcore_barrier()

    @pl.when(cid == 0)
    def _():
        pltpu.sync_copy(acc.at[pl.ds(r0, RPW)], t_out0.at[pl.ds(r0, RPW)])

    @pl.when(cid == 1)
    def _():
        pltpu.sync_copy(acc.at[pl.ds(r0, RPW)], t_out1.at[pl.ds(r0, RPW)])


CB2 = 64
NCH2 = EP // (NW * CB2)


@functools.partial(
    pl.kernel,
    out_type=[jax.ShapeDtypeStruct((NP, D), jnp.float32),
              jax.ShapeDtypeStruct((NP, D), jnp.float32)],
    mesh=_sc_mesh,
    compiler_params=_sc_params,
    scratch_types=[
        pltpu.VMEM_SHARED((NP, D), jnp.float32),
        pltpu.VMEM((4, CB2), jnp.int32),
        pltpu.VMEM((4, CB2), jnp.int32),
        pltpu.VMEM((4, CB2, H), jnp.float32),
        pltpu.VMEM((2, CB2, D), jnp.float32),
        pltpu.VMEM((2, CB2, D), jnp.float32),
    ] + [pltpu.SemaphoreType.DMA] * 16,
)
def _edge_sc2(vn_hbm, dst_hbm, src_hbm, p_hbm, zero_hbm,
              v_out0, v_out1, acc, dstv, srcv, pvr, vv, mv, *sems):
    semd = sems[0:4]
    semsr = sems[4:8]
    semt = sems[8:12]
    semv = sems[12:14]
    semm = sems[14:16]
    cid = lax.axis_index("c")
    sid = lax.axis_index("s")
    w = sid * 2 + cid
    r0 = sid * RPW
    pltpu.sync_copy(zero_hbm.at[pl.ds(r0, RPW)], acc.at[pl.ds(r0, RPW)])
    plsc.subcore_barrier()

    lane = lax.iota(jnp.int32, 16)
    z16 = jnp.zeros((16,), jnp.int32)
    c0 = w * NCH2

    def meta_copies(ci, slot):
        return (pltpu.make_async_copy(dst_hbm.at[c0 + ci], dstv.at[slot],
                                      semd[slot]),
                pltpu.make_async_copy(src_hbm.at[c0 + ci], srcv.at[slot],
                                      semsr[slot]),
                pltpu.make_async_copy(p_hbm.at[pl.ds((c0 + ci) * CB2, CB2)],
                                      pvr.at[slot], semt[slot]))

    def gather_copies(b, slot):
        return (pltpu.make_async_copy(vn_hbm.at[srcv.at[slot]], vv.at[b],
                                      semv[b]),)

    def out_copies(b, slot):
        return (pltpu.make_async_copy(mv.at[b], acc.at[dstv.at[slot]],
                                      semm[b]),)

    for cd in meta_copies(0, 0) + meta_copies(1, 1):
        cd.start()
    _, cs0, _ = meta_copies(0, 0)
    cs0.wait()
    for cg in gather_copies(0, 0):
        cg.start()

    def quad(qi, carry):
        for u in range(4):
            ci = qi * 4 + u
            b = u % 2
            slot_n = (u + 1) % 4
            slot_p = (u + 2) % 4
            for cg in gather_copies(b, u):
                cg.wait()
            if u < 2:
                @pl.when(qi > 0)
                def _(b=b, u=u):
                    for cm in out_copies(b, u):
                        cm.wait()
            else:
                for cm in out_copies(b, u):
                    cm.wait()
            if u < 2:
                for cd in meta_copies(ci + 2, slot_p):
                    cd.start()
            else:
                @pl.when(qi < NCH2 // 4 - 1)
                def _(ci=ci, slot_p=slot_p):
                    for cd in meta_copies(ci + 2, slot_p):
                        cd.start()

            def start_next(slot_n=slot_n, b=b):
                _, csn, _ = meta_copies(0, slot_n)
                csn.wait()
                for cg in gather_copies(1 - b, slot_n):
                    cg.start()
            if u < 3:
                start_next()
            else:
                pl.when(qi < NCH2 // 4 - 1)(start_next)

            _, _, ct = meta_copies(0, u)
            ct.wait()

            def stage_b(e, carry_b, b=b, u=u):
                erow = z16 + e
                for h in range(H):
                    vvec = plsc.load_gather(vv, [z16 + b, erow, h * DK + lane])
                    pb = plsc.load_gather(pvr, [z16 + u, erow, z16 + h])
                    plsc.store_scatter(mv, [z16 + b, erow, h * DK + lane],
                                       vvec * pb)
                return carry_b
            lax.fori_loop(0, CB2, stage_b, 0)

            cdw, _, _ = meta_copies(0, u)
            cdw.wait()
            for cm in out_copies(b, u):
                cm.start(add=True)
        return carry
    lax.fori_loop(0, NCH2 // 4, quad, 0)

    for b in range(2):
        for cm in out_copies(b, 2 + b):
            cm.wait()

    plsc.subcore_barrier()

    @pl.when(cid == 0)
    def _():
        pltpu.sync_copy(acc.at[pl.ds(r0, RPW)], v_out0.at[pl.ds(r0, RPW)])

    @pl.when(cid == 1)
    def _():
        pltpu.sync_copy(acc.at[pl.ds(r0, RPW)], v_out1.at[pl.ds(r0, RPW)])


def kernel(node_attr, batch_idx, edge_index, strats_spd, atom_emb, summary_emb,
           W_spd_enc, Wq, bq, Wk, bk, Wv, bv, Wa, ba, Wspd, Wlin, blin, gn, bn,
           go, bo, Wfin, bfin):
    del batch_idx, summary_emb
    dmat = (atom_emb[:, 1, :] - atom_emb[:, 0, :])
    cvec = jnp.sum(atom_emb[:, 0, :], axis=0)[None, :]
    attr_f = node_attr.astype(jnp.float32)

    src = edge_index[0]
    dst = edge_index[1]

    d_ids = jnp.arange(D)
    c32 = jnp.arange(32)
    r16 = jnp.where((d_ids[None, :] // DK) == jnp.arange(16)[:, None],
                    1.0, 0.0).astype(jnp.float32)

    pad_e = EP - E
    i32 = jnp.int32
    dst_p = jnp.concatenate(
        [dst.astype(i32), jnp.full((pad_e,), N, i32)]).reshape(EP // CB, CB)
    src_p = jnp.concatenate(
        [src.astype(i32), jnp.zeros((pad_e,), i32)]).reshape(EP // CB, CB)
    strat_p = jnp.concatenate(
        [strats_spd, jnp.zeros((pad_e, 4), jnp.float32)]).reshape(EP // CB, CB, 4)
    dst_p2 = dst_p.reshape(EP // CB2, CB2)
    src_p2 = src_p.reshape(EP // CB2, CB2)
    zero_s = jnp.zeros((NP, SW), jnp.float32)
    zero_v = jnp.zeros((NP, D), jnp.float32)

    h = _enc(attr_f, dmat, cvec)
    for l in range(L):
        ck = W_spd_enc @ Wspd[l] @ Wk[l]
        cv = W_spd_enc @ Wspd[l] @ Wv[l]
        mk = jnp.where((c32[None, :] // 4) == (d_ids[:, None] // DK),
                       ck.T[:, c32 % 4], 0.0)
        mv = jnp.where((d_ids[None, :] // DK) == (c32[:, None] // 4),
                       cv[c32 % 4, :], 0.0)
        qcat, kn, vn = _pre(h, Wq[l] * np.float32(0.25),
                            bq[l][None] * np.float32(0.25), Wk[l],
                            bk[l][None], Wv[l], bv[l][None], mk)
        p_e, t0, t1 = _edge_sc1(qcat, kn, dst_p, src_p, strat_p, zero_s)
        v0, v1 = _edge_sc2(vn, dst_p2, src_p2, p_e, zero_v)
        h = _post(v0, v1, t0, t1, h, mv, r16, Wa[l], ba[l][None], gn[l][None],
                  bn[l][None], Wlin[l], blin[l][None], go[l][None], bo[l][None])
    return _fin(h, Wfin, bfin[None])


# speedup vs baseline: 3.4949x; 1.1250x over previous
import functools

import jax
import jax.numpy as jnp
import numpy as np
from jax import lax
from jax.experimental import pallas as pl
from jax.experimental.pallas import tpu as pltpu
from jax.experimental.pallas import tpu_sc as plsc

N = 10000
E = 160000
H = 8
D = 128
DK = D // H
L = 2
NOUT = 128
BN = 1000
AW = 176
QW = D + 32

NP = 10240
QW1 = QW + 1
D1 = D + 1
CB = 128
NCH = 40
NW = 32
EP = NW * NCH * CB
RPW = NP // 16


def _gelu(x):
    return x * 0.5 * (1.0 + jax.lax.erf(x * np.float32(1.0 / np.sqrt(2.0))))


def _ln(x, g, b):
    m = jnp.mean(x, axis=-1, keepdims=True)
    v = jnp.mean((x - m) ** 2, axis=-1, keepdims=True)
    return (x - m) * jax.lax.rsqrt(v + 1e-5) * g + b


def _enc_body(attr_ref, dmat_ref, c_ref, h_ref):
    h_ref[...] = (
        jnp.dot(attr_ref[...], dmat_ref[...], preferred_element_type=jnp.float32)
        + c_ref[...]
    )


def _enc(attr_f, dmat, c):
    return pl.pallas_call(
        _enc_body,
        grid=(N // BN,),
        in_specs=[
            pl.BlockSpec((BN, 9), lambda i: (i, 0)),
            pl.BlockSpec((9, D), lambda i: (0, 0)),
            pl.BlockSpec((1, D), lambda i: (0, 0)),
        ],
        out_specs=pl.BlockSpec((BN, D), lambda i: (i, 0)),
        out_shape=jax.ShapeDtypeStruct((N, D), jnp.float32),
    )(attr_f, dmat, c)


def _pre_body(h_ref, wq_ref, bq_ref, wk_ref, bk_ref, wv_ref, bv_ref, mk_ref,
              qcat_ref, k_ref, v_ref):
    h = h_ref[...]
    qn = jnp.dot(h, wq_ref[...], preferred_element_type=jnp.float32) + bq_ref[...]
    kn = jnp.dot(h, wk_ref[...], preferred_element_type=jnp.float32) + bk_ref[...]
    vn = jnp.dot(h, wv_ref[...], preferred_element_type=jnp.float32) + bv_ref[...]
    gq = jnp.dot(qn, mk_ref[...], preferred_element_type=jnp.float32)
    qcat_ref[:, :D] = qn
    qcat_ref[:, D:QW] = gq
    qcat_ref[:, QW:] = jnp.zeros_like(qcat_ref[:, QW:])
    k_ref[:, :D] = kn
    k_ref[:, D:] = jnp.zeros_like(k_ref[:, D:])
    v_ref[...] = vn


def _pre(h, wq, bq, wk, bk, wv, bv, mk):
    full = lambda r, c: pl.BlockSpec((r, c), lambda i: (0, 0))
    return pl.pallas_call(
        _pre_body,
        grid=(N // BN,),
        in_specs=[
            pl.BlockSpec((BN, D), lambda i: (i, 0)),
            full(D, D), full(1, D), full(D, D), full(1, D), full(D, D),
            full(1, D), full(D, 32),
        ],
        out_specs=[
            pl.BlockSpec((BN, QW1), lambda i: (i, 0)),
            pl.BlockSpec((BN, D1), lambda i: (i, 0)),
            pl.BlockSpec((BN, D), lambda i: (i, 0)),
        ],
        out_shape=[
            jax.ShapeDtypeStruct((NP, QW1), jnp.float32),
            jax.ShapeDtypeStruct((NP, D1), jnp.float32),
            jax.ShapeDtypeStruct((NP, D), jnp.float32),
        ],
    )(h, wq, bq, wk, bk, wv, bv, mk)


def _post_body(v0_ref, v1_ref, t0_ref, t1_ref, h_ref, mv_ref, r16_ref, wa_ref,
               ba_ref, gn_ref, bn_ref, wlin_ref, blin_ref, go_ref, bo_ref,
               hout_ref):
    ts = t0_ref[...] + t1_ref[...]
    num = (v0_ref[...] + v1_ref[...]
           + jnp.dot(ts[:, :32], mv_ref[...],
                     preferred_element_type=jnp.float32))
    srep = jnp.dot(ts[:, 32:], r16_ref[...],
                   preferred_element_type=jnp.float32)
    aggr = num / (srep + 1e-16)
    h = h_ref[...]
    t = (jnp.dot(_gelu(aggr), wa_ref[...], preferred_element_type=jnp.float32)
         + ba_ref[...] + h)
    t = _ln(t, gn_ref[...], bn_ref[...])
    t2 = (jnp.dot(_gelu(t), wlin_ref[...], preferred_element_type=jnp.float32)
          + blin_ref[...] + t)
    hout_ref[...] = _ln(t2, go_ref[...], bo_ref[...])


def _post(v0, v1, t0, t1, h, mv, r16, wa, ba, gn, bn, wlin, blin, go, bo):
    full = lambda r, c: pl.BlockSpec((r, c), lambda i: (0, 0))
    return pl.pallas_call(
        _post_body,
        grid=(N // BN,),
        in_specs=[
            pl.BlockSpec((BN, D), lambda i: (i, 0)),
            pl.BlockSpec((BN, D), lambda i: (i, 0)),
            pl.BlockSpec((BN, SW), lambda i: (i, 0)),
            pl.BlockSpec((BN, SW), lambda i: (i, 0)),
            pl.BlockSpec((BN, D), lambda i: (i, 0)),
            full(32, D), full(16, D), full(D, D), full(1, D), full(1, D),
            full(1, D), full(D, D), full(1, D), full(1, D), full(1, D),
        ],
        out_specs=pl.BlockSpec((BN, D), lambda i: (i, 0)),
        out_shape=jax.ShapeDtypeStruct((N, D), jnp.float32),
    )(v0, v1, t0, t1, h, mv, r16, wa, ba, gn, bn, wlin, blin, go, bo)


def _fin_body(h_ref, w_ref, b_ref, o_ref):
    o_ref[...] = (jnp.dot(h_ref[...], w_ref[...],
                          preferred_element_type=jnp.float32) + b_ref[...])


def _fin(h, w, b):
    return pl.pallas_call(
        _fin_body,
        grid=(N // BN,),
        in_specs=[
            pl.BlockSpec((BN, D), lambda i: (i, 0)),
            pl.BlockSpec((D, NOUT), lambda i: (0, 0)),
            pl.BlockSpec((1, NOUT), lambda i: (0, 0)),
        ],
        out_specs=pl.BlockSpec((BN, NOUT), lambda i: (i, 0)),
        out_shape=jax.ShapeDtypeStruct((N, NOUT), jnp.float32),
    )(h, w, b)


_sc_mesh = plsc.VectorSubcoreMesh(core_axis_name="c", subcore_axis_name="s")
_sc_params = pltpu.CompilerParams(use_tc_tiling_on_sc=False,
                                  needs_layout_passes=False)
SW = 48


@functools.partial(
    pl.kernel,
    out_type=[jax.ShapeDtypeStruct((EP, H + 1), jnp.float32),
              jax.ShapeDtypeStruct((NP, SW), jnp.float32),
              jax.ShapeDtypeStruct((NP, SW), jnp.float32)],
    mesh=_sc_mesh,
    compiler_params=_sc_params,
    scratch_types=[
        pltpu.VMEM_SHARED((NP, SW), jnp.float32),
        pltpu.VMEM((4, CB), jnp.int32),
        pltpu.VMEM((4, CB), jnp.int32),
        pltpu.VMEM((4, CB, 5), jnp.float32),
        pltpu.VMEM((2, CB, QW1), jnp.float32),
        pltpu.VMEM((2, CB, D1), jnp.float32),
        pltpu.VMEM((2, CB, H + 1), jnp.float32),
        pltpu.VMEM((2, CB, SW), jnp.float32),
    ] + [pltpu.SemaphoreType.DMA] * 20,
)
def _edge_sc1(qcat_hbm, kn_hbm, dst_hbm, src_hbm, strat_hbm, zero_hbm,
              p_out, t_out0, t_out1, acc, dstv, srcv, stratv, qv, kv, pv, mv,
              *sems):
    semd = sems[0:4]
    semsr = sems[4:8]
    semt = sems[8:12]
    semq = sems[12:14]
    semk = sems[14:16]
    semp = sems[16:18]
    semm = sems[18:20]
    cid = lax.axis_index("c")
    sid = lax.axis_index("s")
    w = sid * 2 + cid
    r0 = sid * RPW
    pltpu.sync_copy(zero_hbm.at[pl.ds(r0, RPW)], acc.at[pl.ds(r0, RPW)])
    plsc.subcore_barrier()

    lane = lax.iota(jnp.int32, 16)
    z16 = jnp.zeros((16,), jnp.int32)
    c0 = w * NCH

    def meta_copies(ci, slot):
        return (pltpu.make_async_copy(dst_hbm.at[c0 + ci], dstv.at[slot],
                                      semd[slot]),
                pltpu.make_async_copy(src_hbm.at[c0 + ci], srcv.at[slot],
                                      semsr[slot]),
                pltpu.make_async_copy(strat_hbm.at[c0 + ci], stratv.at[slot],
                                      semt[slot]))

    def gather_copies(b, slot):
        return (pltpu.make_async_copy(qcat_hbm.at[dstv.at[slot]], qv.at[b],
                                      semq[b]),
                pltpu.make_async_copy(kn_hbm.at[srcv.at[slot]], kv.at[b],
                                      semk[b]))

    def out_copies(ci, b, slot):
        return (pltpu.make_async_copy(
                    pv.at[b], p_out.at[pl.ds((c0 + ci) * CB, CB)], semp[b]),
                pltpu.make_async_copy(mv.at[b], acc.at[dstv.at[slot]],
                                      semm[b]))

    for cd in meta_copies(0, 0) + meta_copies(1, 1):
        cd.start()
    cd0, cs0, _ = meta_copies(0, 0)
    cd0.wait()
    cs0.wait()
    for cg in gather_copies(0, 0):
        cg.start()

    def quad(qi, carry):
        for u in range(4):
            ci = qi * 4 + u
            b = u % 2
            slot_n = (u + 1) % 4
            slot_p = (u + 2) % 4
            for cg in gather_copies(b, u):
                cg.wait()
            if u < 2:
                @pl.when(qi > 0)
                def _(b=b, u=u):
                    cp, cm = out_copies(0, b, u)
                    cp.wait()
                    cm.wait()
            else:
                cp, cm = out_copies(0, b, u)
                cp.wait()
                cm.wait()
            if u < 2:
                for cd in meta_copies(ci + 2, slot_p):
                    cd.start()
            else:
                @pl.when(qi < NCH // 4 - 1)
                def _(ci=ci, slot_p=slot_p):
                    for cd in meta_copies(ci + 2, slot_p):
                        cd.start()
            def start_next(slot_n=slot_n, b=b):
                cdn, csn, _ = meta_copies(0, slot_n)
                cdn.wait()
                csn.wait()
                for cg in gather_copies(1 - b, slot_n):
                    cg.start()
            if u < 3:
                start_next()
            else:
                pl.when(qi < NCH // 4 - 1)(start_next)
            _, _, ct = meta_copies(0, u)
            ct.wait()

            def stage_a(g, carry_a, b=b, u=u):
                erow = g * 16 + lane
                svecs = [plsc.load_gather(stratv, [z16 + u, erow, z16 + j])
                         for j in range(4)]

                def head(h, carry_h, b=b, erow=erow, svecs=svecs):
                    accs = [jnp.zeros((16,), jnp.float32) for _ in range(4)]
                    for dk in range(DK):
                        col = z16 + (h * DK + dk)
                        accs[dk % 4] = accs[dk % 4] + (
                            plsc.load_gather(qv, [z16 + b, erow, col])
                            * plsc.load_gather(kv, [z16 + b, erow, col]))
                    for j in range(4):
                        gq = plsc.load_gather(
                            qv, [z16 + b, erow, z16 + (D + h * 4 + j)])
                        accs[j] = accs[j] + gq * svecs[j]
                    a = (accs[0] + accs[1]) + (accs[2] + accs[3])
                    plsc.store_scatter(pv, [z16 + b, erow, z16 + h],
                                       jnp.exp(a))
                    return carry_h
                lax.fori_loop(0, H, head, 0)
                return carry_a
            lax.fori_loop(0, CB // 16, stage_a, 0)

            def stage_b(e, carry_b, b=b, u=u):
                erow = z16 + e
                jj = lane % 4
                hh = lane // 4
                sb = plsc.load_gather(stratv, [z16 + u, erow, jj])
                pb0 = plsc.load_gather(pv, [z16 + b, erow, hh])
                plsc.store_scatter(mv, [z16 + b, erow, lane], pb0 * sb)
                pb1 = plsc.load_gather(pv, [z16 + b, erow, 4 + hh])
                plsc.store_scatter(mv, [z16 + b, erow, 16 + lane], pb1 * sb)
                ps = plsc.load_gather(pv, [z16 + b, erow, jnp.minimum(lane, 7)])
                ps = jnp.where(lane < 8, ps, jnp.float32(0.0))
                plsc.store_scatter(mv, [z16 + b, erow, 32 + lane], ps)
                return carry_b
            lax.fori_loop(0, CB, stage_b, 0)

            cp, cm = out_copies(ci, b, u)
            cp.start()
            cm.start(add=True)
        return carry
    lax.fori_loop(0, NCH // 4, quad, 0)

    for b in range(2):
        cp, cm = out_copies(0, b, 2 + b)
        cp.wait()
        cm.wait()

    plsc.subcore_barrier()

    @pl.when(cid == 0)
    def _():
        pltpu.sync_copy(acc.at[pl.ds(r0, RPW)], t_out0.at[pl.ds(r0, RPW)])

    @pl.when(cid == 1)
    def _():
        pltpu.sync_copy(acc.at[pl.ds(r0, RPW)], t_out1.at[pl.ds(r0, RPW)])


CB2 = 64
NCH2 = EP // (NW * CB2)


@functools.partial(
    pl.kernel,
    out_type=[jax.ShapeDtypeStruct((NP, D), jnp.float32),
              jax.ShapeDtypeStruct((NP, D), jnp.float32)],
    mesh=_sc_mesh,
    compiler_params=_sc_params,
    scratch_types=[
        pltpu.VMEM_SHARED((NP, D), jnp.float32),
        pltpu.VMEM((4, CB2), jnp.int32),
        pltpu.VMEM((4, CB2), jnp.int32),
        pltpu.VMEM((4, CB2, H + 1), jnp.float32),
        pltpu.VMEM((2, CB2, D), jnp.float32),
        pltpu.VMEM((2, CB2, D), jnp.float32),
    ] + [pltpu.SemaphoreType.DMA] * 16,
)
def _edge_sc2(vn_hbm, dst_hbm, src_hbm, p_hbm, zero_hbm,
              v_out0, v_out1, acc, dstv, srcv, pvr, vv, mv, *sems):
    semd = sems[0:4]
    semsr = sems[4:8]
    semt = sems[8:12]
    semv = sems[12:14]
    semm = sems[14:16]
    cid = lax.axis_index("c")
    sid = lax.axis_index("s")
    w = sid * 2 + cid
    r0 = sid * RPW
    pltpu.sync_copy(zero_hbm.at[pl.ds(r0, RPW)], acc.at[pl.ds(r0, RPW)])
    plsc.subcore_barrier()

    lane = lax.iota(jnp.int32, 16)
    z16 = jnp.zeros((16,), jnp.int32)
    c0 = w * NCH2

    def meta_copies(ci, slot):
        return (pltpu.make_async_copy(dst_hbm.at[c0 + ci], dstv.at[slot],
                                      semd[slot]),
                pltpu.make_async_copy(src_hbm.at[c0 + ci], srcv.at[slot],
                                      semsr[slot]),
                pltpu.make_async_copy(p_hbm.at[pl.ds((c0 + ci) * CB2, CB2)],
                                      pvr.at[slot], semt[slot]))

    def gather_copies(b, slot):
        return (pltpu.make_async_copy(vn_hbm.at[srcv.at[slot]], vv.at[b],
                                      semv[b]),)

    def out_copies(b, slot):
        return (pltpu.make_async_copy(mv.at[b], acc.at[dstv.at[slot]],
                                      semm[b]),)

    for cd in meta_copies(0, 0) + meta_copies(1, 1):
        cd.start()
    _, cs0, _ = meta_copies(0, 0)
    cs0.wait()
    for cg in gather_copies(0, 0):
        cg.start()

    def quad(qi, carry):
        for u in range(4):
            ci = qi * 4 + u
            b = u % 2
            slot_n = (u + 1) % 4
            slot_p = (u + 2) % 4
            for cg in gather_copies(b, u):
                cg.wait()
            if u < 2:
                @pl.when(qi > 0)
                def _(b=b, u=u):
                    for cm in out_copies(b, u):
                        cm.wait()
            else:
                for cm in out_copies(b, u):
                    cm.wait()
            if u < 2:
                for cd in meta_copies(ci + 2, slot_p):
                    cd.start()
            else:
                @pl.when(qi < NCH2 // 4 - 1)
                def _(ci=ci, slot_p=slot_p):
                    for cd in meta_copies(ci + 2, slot_p):
                        cd.start()

            def start_next(slot_n=slot_n, b=b):
                _, csn, _ = meta_copies(0, slot_n)
                csn.wait()
                for cg in gather_copies(1 - b, slot_n):
                    cg.start()
            if u < 3:
                start_next()
            else:
                pl.when(qi < NCH2 // 4 - 1)(start_next)

            _, _, ct = meta_copies(0, u)
            ct.wait()

            def stage_b(e, carry_b, b=b, u=u):
                erow = z16 + e
                for h in range(H):
                    vvec = plsc.load_gather(vv, [z16 + b, erow, h * DK + lane])
                    pb = plsc.load_gather(pvr, [z16 + u, erow, z16 + h])
                    plsc.store_scatter(mv, [z16 + b, erow, h * DK + lane],
                                       vvec * pb)
                return carry_b
            lax.fori_loop(0, CB2, stage_b, 0)

            cdw, _, _ = meta_copies(0, u)
            cdw.wait()
            for cm in out_copies(b, u):
                cm.start(add=True)
        return carry
    lax.fori_loop(0, NCH2 // 4, quad, 0)

    for b in range(2):
        for cm in out_copies(b, 2 + b):
            cm.wait()

    plsc.subcore_barrier()

    @pl.when(cid == 0)
    def _():
        pltpu.sync_copy(acc.at[pl.ds(r0, RPW)], v_out0.at[pl.ds(r0, RPW)])

    @pl.when(cid == 1)
    def _():
        pltpu.sync_copy(acc.at[pl.ds(r0, RPW)], v_out1.at[pl.ds(r0, RPW)])


def kernel(node_attr, batch_idx, edge_index, strats_spd, atom_emb, summary_emb,
           W_spd_enc, Wq, bq, Wk, bk, Wv, bv, Wa, ba, Wspd, Wlin, blin, gn, bn,
           go, bo, Wfin, bfin):
    del batch_idx, summary_emb
    dmat = (atom_emb[:, 1, :] - atom_emb[:, 0, :])
    cvec = jnp.sum(atom_emb[:, 0, :], axis=0)[None, :]
    attr_f = node_attr.astype(jnp.float32)

    src = edge_index[0]
    dst = edge_index[1]

    d_ids = jnp.arange(D)
    c32 = jnp.arange(32)
    r16 = jnp.where((d_ids[None, :] // DK) == jnp.arange(16)[:, None],
                    1.0, 0.0).astype(jnp.float32)

    pad_e = EP - E
    i32 = jnp.int32
    dst_p = jnp.concatenate(
        [dst.astype(i32), jnp.full((pad_e,), N, i32)]).reshape(EP // CB, CB)
    src_p = jnp.concatenate(
        [src.astype(i32), jnp.zeros((pad_e,), i32)]).reshape(EP // CB, CB)
    strat_p = jnp.pad(
        jnp.concatenate([strats_spd, jnp.zeros((pad_e, 4), jnp.float32)]),
        ((0, 0), (0, 1))).reshape(EP // CB, CB, 5)
    dst_p2 = dst_p.reshape(EP // CB2, CB2)
    src_p2 = src_p.reshape(EP // CB2, CB2)
    zero_s = jnp.zeros((NP, SW), jnp.float32)
    zero_v = jnp.zeros((NP, D), jnp.float32)

    h = _enc(attr_f, dmat, cvec)
    for l in range(L):
        ck = W_spd_enc @ Wspd[l] @ Wk[l]
        cv = W_spd_enc @ Wspd[l] @ Wv[l]
        mk = jnp.where((c32[None, :] // 4) == (d_ids[:, None] // DK),
                       ck.T[:, c32 % 4], 0.0)
        mv = jnp.where((d_ids[None, :] // DK) == (c32[:, None] // 4),
                       cv[c32 % 4, :], 0.0)
        qcat, kn, vn = _pre(h, Wq[l] * np.float32(0.25),
                            bq[l][None] * np.float32(0.25), Wk[l],
                            bk[l][None], Wv[l], bv[l][None], mk)
        p_e, t0, t1 = _edge_sc1(qcat, kn, dst_p, src_p, strat_p, zero_s)
        v0, v1 = _edge_sc2(vn, dst_p2, src_p2, p_e, zero_v)
        h = _post(v0, v1, t0, t1, h, mv, r16, Wa[l], ba[l][None], gn[l][None],
                  bn[l][None], Wlin[l], blin[l][None], go[l][None], bo[l][None])
    return _fin(h, Wfin, bfin[None])
